# Initial kernel scaffold; baseline (speedup 1.0000x reference)
import jax
import jax.numpy as jnp
from jax.experimental import pallas as pl


def kernel(x_user, x_item, edge_index_buys, edge_index_views, edge_index_rev):
    raise NotImplementedError("write your pallas kernel here")


# speedup vs baseline: 8.3733x; 8.3733x over previous
import jax
import jax.numpy as jnp
from jax import lax
from jax.experimental import pallas as pl
from jax.experimental.pallas import tpu as pltpu
from jax.experimental.pallas import tpu_sc as plsc

N = 10000
D = 128
E = 160000
NS = 16
CH = 80
GC = 25
NG = 5
FB = 8
CW = 16
NK = D // 16


def _body(x_user, x_item, idx_a, idx_b, idx_r,
          out_user, out_item,
          acc, cnt, rows0, rows1, idx_v, fcnt, ones,
          sem0, sem1):
    core = lax.axis_index("c")
    sub = lax.axis_index("s")
    fbase = sub * (FB * CH)

    zero16 = jnp.zeros((16,), jnp.float32)
    one16 = jnp.ones((16,), jnp.float32)

    def init_ones(r, carry):
        ones[r, pl.ds(0, CW)] = one16
        return carry

    lax.fori_loop(0, CH, init_ones, 0)

    bufs = (rows0, rows1)
    sems = (sem0, sem1)

    def gather(x_hbm, cc, b):
        return pltpu.make_async_copy(
            x_hbm.at[idx_v.at[2 * cc]], bufs[b], sems[b])

    def drain(x_hbm, cc, b):
        g = gather(x_hbm, cc, b)
        g.wait()
        pltpu.sync_copy(bufs[b], acc.at[idx_v.at[2 * cc + 1]], add=True)
        pltpu.sync_copy(ones, cnt.at[idx_v.at[2 * cc + 1]], add=True)

    def conv_loop(x_hbm, idx_hbm):
        for g in range(NG):
            pltpu.sync_copy(idx_hbm.at[sub, g], idx_v)
            for b in range(2):
                gather(x_hbm, b, b).start()

            def pair(i, carry):
                for b in range(2):
                    cc = 2 * i + b
                    drain(x_hbm, cc, b)
                    gather(x_hbm, cc + 2, b).start()
                return carry

            lax.fori_loop(0, GC // 2 - 1, pair, 0)

            drain(x_hbm, GC - 3, 0)
            gather(x_hbm, GC - 1, 0).start()
            drain(x_hbm, GC - 2, 1)
            drain(x_hbm, GC - 1, 0)

    def zero_slices():
        def zrow(r, carry):
            for k in range(NK):
                rows0[r, pl.ds(k * 16, 16)] = zero16
            fcnt[r, pl.ds(0, CW)] = zero16
            return carry

        lax.fori_loop(0, CH, zrow, 0)
        for c in range(FB):
            rowbase = fbase + c * CH

            @pl.when(rowbase < N)
            def _():
                pltpu.sync_copy(rows0, acc.at[pl.ds(rowbase, CH)])
                pltpu.sync_copy(fcnt, cnt.at[pl.ds(rowbase, CH)])

    def finalize(out_hbm, add_prev):
        for c in range(FB):
            rowbase = fbase + c * CH

            @pl.when(rowbase < N)
            def _():
                pltpu.sync_copy(acc.at[pl.ds(rowbase, CH)], rows0)
                pltpu.sync_copy(cnt.at[pl.ds(rowbase, CH)], fcnt)
                if add_prev:
                    pltpu.sync_copy(out_hbm.at[pl.ds(rowbase, CH)], rows1)

                def row_fn(r, carry):
                    cv = fcnt[r, pl.ds(0, CW)]
                    scale = 1.0 / jnp.maximum(cv, 1.0)
                    for k in range(NK):
                        v = rows0[r, pl.ds(k * 16, 16)] * scale
                        if add_prev:
                            v = v + rows1[r, pl.ds(k * 16, 16)]
                        rows0[r, pl.ds(k * 16, 16)] = v
                    return carry

                lax.fori_loop(0, CH, row_fn, 0)
                pltpu.sync_copy(rows0, out_hbm.at[pl.ds(rowbase, CH)])

    for p in range(2):
        if p == 0:
            zero_slices()
        else:
            @pl.when(core == 0)
            def _():
                zero_slices()

        plsc.subcore_barrier()

        if p == 0:
            @pl.when(core == 0)
            def _():
                conv_loop(x_user, idx_a)

            @pl.when(core == 1)
            def _():
                conv_loop(x_item, idx_r)
        else:
            @pl.when(core == 0)
            def _():
                conv_loop(x_user, idx_b)

        plsc.subcore_barrier()

        if p == 0:
            @pl.when(core == 0)
            def _():
                finalize(out_item, add_prev=False)

            @pl.when(core == 1)
            def _():
                finalize(out_user, add_prev=False)
        else:
            @pl.when(core == 0)
            def _():
                finalize(out_item, add_prev=True)


@jax.jit
def kernel(x_user, x_item, edge_index_buys, edge_index_views, edge_index_rev):
    def shape_idx(e):
        e = e.astype(jnp.int32).reshape(2, NS, NG, GC, CH)
        e = jnp.transpose(e, (1, 2, 3, 0, 4))
        return e.reshape(NS, NG, 2 * GC, CH)

    args = [x_user, x_item,
            shape_idx(edge_index_buys),
            shape_idx(edge_index_views),
            shape_idx(edge_index_rev)]

    mesh = plsc.VectorSubcoreMesh(core_axis_name="c", subcore_axis_name="s",
                                  num_cores=2, num_subcores=NS)
    f = pl.kernel(
        _body,
        out_type=(
            jax.ShapeDtypeStruct((N, D), jnp.float32),
            jax.ShapeDtypeStruct((N, D), jnp.float32),
        ),
        mesh=mesh,
        scratch_types=[
            pltpu.VMEM_SHARED((N, D), jnp.float32),
            pltpu.VMEM_SHARED((N, CW), jnp.float32),
            pltpu.VMEM((CH, D), jnp.float32),
            pltpu.VMEM((CH, D), jnp.float32),
            pltpu.VMEM((2 * GC, CH), jnp.int32),
            pltpu.VMEM((CH, CW), jnp.float32),
            pltpu.VMEM((CH, CW), jnp.float32),
            pltpu.SemaphoreType.DMA,
            pltpu.SemaphoreType.DMA,
        ],
        compiler_params=pltpu.CompilerParams(use_tc_tiling_on_sc=False),
        name="hetero_routing_sc",
    )
    out_user, out_item = f(*args)
    return (out_user, out_item)


# speedup vs baseline: 8.9059x; 1.0636x over previous
import jax
import jax.numpy as jnp
from jax import lax
from jax.experimental import pallas as pl
from jax.experimental.pallas import tpu as pltpu
from jax.experimental.pallas import tpu_sc as plsc

N = 10000
D = 128
E = 160000
NS = 16
CH = 80
CHV = 40
GC = 25
NG = 5
FB = 8
CW = 16
NK = D // 16


def _body(x_user, x_item, src_a, dst_a, src_r, dst_r, src_v, dst_v,
          out_user, buys_mean, pacc, pcnt,
          acc, cnt, rows0, rows1, vrows0, vrows1,
          src_i, dst_i, vsrc_i, vdst_i, fcnt, ones, vones,
          sem0, sem1):
    core = lax.axis_index("c")
    sub = lax.axis_index("s")
    fbase = sub * (FB * CH)

    zero16 = jnp.zeros((16,), jnp.float32)
    one16 = jnp.ones((16,), jnp.float32)

    def init_ones(r, carry):
        ones[r, pl.ds(0, CW)] = one16
        return carry

    lax.fori_loop(0, CH, init_ones, 0)

    def init_vones(r, carry):
        vones[r, pl.ds(0, CW)] = one16
        return carry

    lax.fori_loop(0, CHV, init_vones, 0)

    def conv_loop(x_hbm, src_hbm, dst_hbm, bufs, sbuf, dbuf, onesbuf,
                  core_split):
        for g in range(NG):
            if core_split:
                pltpu.sync_copy(src_hbm.at[core, sub, g], sbuf)
                pltpu.sync_copy(dst_hbm.at[core, sub, g], dbuf)
            else:
                pltpu.sync_copy(src_hbm.at[sub, g], sbuf)
                pltpu.sync_copy(dst_hbm.at[sub, g], dbuf)

            def gather(cc, b):
                return pltpu.make_async_copy(
                    x_hbm.at[sbuf.at[cc]], bufs[b], (sem0, sem1)[b])

            def drain(cc, b):
                gather(cc, b).wait()
                pltpu.sync_copy(bufs[b], acc.at[dbuf.at[cc]], add=True)
                pltpu.sync_copy(onesbuf, cnt.at[dbuf.at[cc]], add=True)

            for b in range(2):
                gather(b, b).start()

            def pair(i, carry):
                for b in range(2):
                    cc = 2 * i + b
                    drain(cc, b)
                    gather(cc + 2, b).start()
                return carry

            lax.fori_loop(0, GC // 2 - 1, pair, 0)

            drain(GC - 3, 0)
            gather(GC - 1, 0).start()
            drain(GC - 2, 1)
            drain(GC - 1, 0)

    def zero_slices():
        def zrow(r, carry):
            for k in range(NK):
                rows0[r, pl.ds(k * 16, 16)] = zero16
            fcnt[r, pl.ds(0, CW)] = zero16
            return carry

        lax.fori_loop(0, CH, zrow, 0)
        for c in range(FB):
            rowbase = fbase + c * CH

            @pl.when(rowbase < N)
            def _():
                pltpu.sync_copy(rows0, acc.at[pl.ds(rowbase, CH)])
                pltpu.sync_copy(fcnt, cnt.at[pl.ds(rowbase, CH)])

    def finalize(out_hbm):
        for c in range(FB):
            rowbase = fbase + c * CH

            @pl.when(rowbase < N)
            def _():
                pltpu.sync_copy(acc.at[pl.ds(rowbase, CH)], rows0)
                pltpu.sync_copy(cnt.at[pl.ds(rowbase, CH)], fcnt)

                def row_fn(r, carry):
                    cv = fcnt[r, pl.ds(0, CW)]
                    scale = 1.0 / jnp.maximum(cv, 1.0)
                    for k in range(NK):
                        rows0[r, pl.ds(k * 16, 16)] = (
                            rows0[r, pl.ds(k * 16, 16)] * scale)
                    return carry

                lax.fori_loop(0, CH, row_fn, 0)
                pltpu.sync_copy(rows0, out_hbm.at[pl.ds(rowbase, CH)])

    def dump_partials():
        for c in range(FB):
            rowbase = fbase + c * CH

            @pl.when(rowbase < N)
            def _():
                pltpu.sync_copy(acc.at[pl.ds(rowbase, CH)], rows0)
                pltpu.sync_copy(rows0, pacc.at[core, pl.ds(rowbase, CH)])
                pltpu.sync_copy(cnt.at[pl.ds(rowbase, CH)], fcnt)
                pltpu.sync_copy(fcnt, pcnt.at[core, pl.ds(rowbase, CH)])

    zero_slices()
    plsc.subcore_barrier()

    @pl.when(core == 0)
    def _():
        conv_loop(x_user, src_a, dst_a, (rows0, rows1), src_i, dst_i,
                  ones, False)

    @pl.when(core == 1)
    def _():
        conv_loop(x_item, src_r, dst_r, (rows0, rows1), src_i, dst_i,
                  ones, False)

    plsc.subcore_barrier()

    @pl.when(core == 0)
    def _():
        finalize(buys_mean)

    @pl.when(core == 1)
    def _():
        finalize(out_user)

    zero_slices()
    plsc.subcore_barrier()
    conv_loop(x_user, src_v, dst_v, (vrows0, vrows1), vsrc_i, vdst_i,
              vones, True)
    plsc.subcore_barrier()
    dump_partials()


def _combine_body(bm_ref, pacc_ref, pcnt_ref, out_ref):
    s = pacc_ref[0] + pacc_ref[1]
    c = pcnt_ref[0][:, :1] + pcnt_ref[1][:, :1]
    out_ref[...] = bm_ref[...] + s / jnp.maximum(c, 1.0)


@jax.jit
def kernel(x_user, x_item, edge_index_buys, edge_index_views, edge_index_rev):
    def full_idx(e):
        e = e.astype(jnp.int32)
        return (e[0].reshape(NS, NG, GC, CH), e[1].reshape(NS, NG, GC, CH))

    def split_idx(e):
        e = e.astype(jnp.int32)
        return (e[0].reshape(2, NS, NG, GC, CHV),
                e[1].reshape(2, NS, NG, GC, CHV))

    src_a, dst_a = full_idx(edge_index_buys)
    src_r, dst_r = full_idx(edge_index_rev)
    src_v, dst_v = split_idx(edge_index_views)

    mesh = plsc.VectorSubcoreMesh(core_axis_name="c", subcore_axis_name="s",
                                  num_cores=2, num_subcores=NS)
    f = pl.kernel(
        _body,
        out_type=(
            jax.ShapeDtypeStruct((N, D), jnp.float32),
            jax.ShapeDtypeStruct((N, D), jnp.float32),
            jax.ShapeDtypeStruct((2, N, D), jnp.float32),
            jax.ShapeDtypeStruct((2, N, CW), jnp.float32),
        ),
        mesh=mesh,
        scratch_types=[
            pltpu.VMEM_SHARED((N, D), jnp.float32),
            pltpu.VMEM_SHARED((N, CW), jnp.float32),
            pltpu.VMEM((CH, D), jnp.float32),
            pltpu.VMEM((CH, D), jnp.float32),
            pltpu.VMEM((CHV, D), jnp.float32),
            pltpu.VMEM((CHV, D), jnp.float32),
            pltpu.VMEM((GC, CH), jnp.int32),
            pltpu.VMEM((GC, CH), jnp.int32),
            pltpu.VMEM((GC, CHV), jnp.int32),
            pltpu.VMEM((GC, CHV), jnp.int32),
            pltpu.VMEM((CH, CW), jnp.float32),
            pltpu.VMEM((CH, CW), jnp.float32),
            pltpu.VMEM((CHV, CW), jnp.float32),
            pltpu.SemaphoreType.DMA,
            pltpu.SemaphoreType.DMA,
        ],
        compiler_params=pltpu.CompilerParams(use_tc_tiling_on_sc=False),
        name="hetero_routing_sc",
    )
    out_user, buys_mean, pacc, pcnt = f(x_user, x_item, src_a, dst_a,
                                        src_r, dst_r, src_v, dst_v)

    BR = 1000
    out_item = pl.pallas_call(
        _combine_body,
        grid=(N // BR,),
        in_specs=[
            pl.BlockSpec((BR, D), lambda i: (i, 0)),
            pl.BlockSpec((2, BR, D), lambda i: (0, i, 0)),
            pl.BlockSpec((2, BR, CW), lambda i: (0, i, 0)),
        ],
        out_specs=pl.BlockSpec((BR, D), lambda i: (i, 0)),
        out_shape=jax.ShapeDtypeStruct((N, D), jnp.float32),
        name="hetero_routing_combine",
    )(buys_mean, pacc, pcnt)

    return (out_user, out_item)
